# baseline (speedup 1.0000x reference)
import functools

import jax
import jax.numpy as jnp
from jax import lax
from jax.experimental import pallas as pl
from jax.experimental.pallas import tpu as pltpu
from jax.experimental.pallas import tpu_sc as plsc

N = 10000
E = 320000
D = 128

NC = 2
NS = 16
NW = NC * NS
EPW = E // NW
K = 80
NCHUNK = EPW // K
DEG_CH = 624

_mesh = plsc.VectorSubcoreMesh(
    core_axis_name="c", subcore_axis_name="s", num_cores=NC, num_subcores=NS
)


@functools.partial(
    pl.kernel,
    out_type=jax.ShapeDtypeStruct((NC * N,), jnp.float32),
    mesh=_mesh,
    scratch_types=[
        pltpu.VMEM((EPW,), jnp.int32),
        pltpu.VMEM((EPW,), jnp.float32),
        pltpu.VMEM((K,), jnp.int32),
        pltpu.VMEM((K,), jnp.int32),
        pltpu.VMEM((DEG_CH + 16,), jnp.float32),
        pltpu.VMEM_SHARED((N,), jnp.float32),
        pltpu.SemaphoreType.DMA,
        pltpu.SemaphoreType.DMA,
    ],
)
def _sc_deg(dst_hbm, ew_hbm, out_hbm, di_all, ew_all, di_a, di_b, zbuf, acc,
            sem_a, sem_b):
    c = lax.axis_index("c")
    s = lax.axis_index("s")
    base = (c * NS + s) * EPW

    pltpu.sync_copy(dst_hbm.at[pl.ds(base, EPW)], di_all)
    pltpu.sync_copy(ew_hbm.at[pl.ds(base, EPW)], ew_all)

    for i in range((DEG_CH + 16) // 16):
        zbuf[pl.ds(i * 16, 16)] = jnp.zeros((16,), jnp.float32)
    pltpu.sync_copy(zbuf.at[pl.ds(0, DEG_CH)], acc.at[pl.ds(s * DEG_CH, DEG_CH)])

    @pl.when(s == NS - 1)
    def _():
        pltpu.sync_copy(zbuf.at[pl.ds(0, 16)], acc.at[pl.ds(N - 16, 16)])

    plsc.subcore_barrier()

    dis = (di_a, di_b)
    sems = (sem_a, sem_b)

    def stage_and_fire(j, par):
        di_v = dis[par]
        for g in range(K // 16):
            di_v[pl.ds(g * 16, 16)] = di_all[pl.ds(j * K + g * 16, 16)]
        pltpu.async_copy(ew_all.at[pl.ds(j * K, K)], acc.at[di_v],
                         sems[par], add=True)

    def drain(j, par):
        di_v = dis[par]
        pltpu.make_async_copy(ew_all.at[pl.ds(j * K, K)], acc.at[di_v],
                              sems[par]).wait()

    stage_and_fire(0, 0)

    def body2(jj, carry):
        stage_and_fire(jj * 2 + 1, 1)
        drain(jj * 2, 0)
        stage_and_fire(jj * 2 + 2, 0)
        drain(jj * 2 + 1, 1)
        return carry

    lax.fori_loop(0, (NCHUNK - 1) // 2, body2, 0)
    drain(NCHUNK - 1, 0)

    plsc.subcore_barrier()

    pltpu.sync_copy(acc.at[pl.ds(s * DEG_CH, DEG_CH)], zbuf.at[pl.ds(0, DEG_CH)])
    pltpu.sync_copy(
        zbuf.at[pl.ds(0, DEG_CH)],
        out_hbm.at[pl.ds(c * N + s * DEG_CH, DEG_CH)],
    )

    @pl.when(s == NS - 1)
    def _():
        pltpu.sync_copy(acc.at[pl.ds(N - 16, 16)], zbuf.at[pl.ds(0, 16)])
        pltpu.sync_copy(
            zbuf.at[pl.ds(0, 16)], out_hbm.at[pl.ds(c * N + N - 16, 16)]
        )


PH_A = 63
PH_B = NCHUNK - PH_A
SI_LEN = PH_A * K


@functools.partial(
    pl.kernel,
    out_type=jax.ShapeDtypeStruct((NC, N, D), jnp.float32),
    mesh=_mesh,
    scratch_types=[
        pltpu.VMEM((SI_LEN,), jnp.int32),
        pltpu.VMEM((K,), jnp.int32),
        pltpu.VMEM((K,), jnp.int32),
        pltpu.VMEM((K,), jnp.float32),
        pltpu.VMEM((K,), jnp.float32),
        pltpu.VMEM((K, D), jnp.float32),
        pltpu.VMEM((K, D), jnp.float32),
        pltpu.VMEM((K, D), jnp.float32),
        pltpu.VMEM((K, D), jnp.float32),
        pltpu.VMEM((24, D), jnp.float32),
        pltpu.VMEM_SHARED((N, D), jnp.float32),
        pltpu.SemaphoreType.DMA,
        pltpu.SemaphoreType.DMA,
        pltpu.SemaphoreType.DMA,
        pltpu.SemaphoreType.DMA,
        pltpu.SemaphoreType.DMA,
        pltpu.SemaphoreType.DMA,
        pltpu.SemaphoreType.DMA,
        pltpu.SemaphoreType.DMA,
        pltpu.SemaphoreType.DMA,
        pltpu.SemaphoreType.DMA,
    ],
)
def _sc_agg(y_hbm, src_hbm, dst_hbm, ew_hbm, out_hbm, si_all,
            di_a, di_b, ew_a, ew_b, buf_0, buf_1, buf_2, buf_3,
            zbuf, acc, gs_0, gs_1, gs_2, gs_3, ss_0, ss_1, ss_2, ss_3,
            ds_a, ds_b):
    c = lax.axis_index("c")
    s = lax.axis_index("s")
    base = (c * NS + s) * EPW

    pltpu.sync_copy(src_hbm.at[pl.ds(base, SI_LEN)], si_all)

    def zrow(i, carry):
        for t in range(D // 16):
            zbuf[i, pl.ds(t * 16, 16)] = jnp.zeros((16,), jnp.float32)
        return carry

    lax.fori_loop(0, 24, zrow, 0)
    for r in range(DEG_CH // 24):
        pltpu.sync_copy(zbuf, acc.at[pl.ds(s * DEG_CH + r * 24, 24)])

    @pl.when(s == NS - 1)
    def _():
        pltpu.sync_copy(zbuf.at[pl.ds(0, 16)], acc.at[pl.ds(N - 16, 16)])

    plsc.subcore_barrier()

    bufs = (buf_0, buf_1, buf_2, buf_3)
    gsems = (gs_0, gs_1, gs_2, gs_3)
    ssems = (ss_0, ss_1, ss_2, ss_3)
    dis = (di_a, di_b)
    ews = (ew_a, ew_b)
    dsems = (ds_a, ds_b)
    H = K // 2

    def emit_phase(nchunks, jbase):
        def fire_g(j, par):
            pltpu.async_copy(
                y_hbm.at[si_all.at[pl.ds(j * K, H)]],
                bufs[par].at[pl.ds(0, H)], gsems[par])
            pltpu.async_copy(
                y_hbm.at[si_all.at[pl.ds(j * K + H, H)]],
                bufs[par].at[pl.ds(H, H)], gsems[par])

        def wait_g(j, par):
            pltpu.make_async_copy(
                y_hbm.at[si_all.at[pl.ds(j * K, K)]], bufs[par], gsems[par]
            ).wait()

        def fire_i(j, ipar):
            off = base + (jbase + j) * K
            pltpu.async_copy(dst_hbm.at[pl.ds(off, K)], dis[ipar],
                             dsems[ipar])
            pltpu.async_copy(ew_hbm.at[pl.ds(off, K)], ews[ipar],
                             dsems[ipar])

        def wait_i(j, ipar):
            off = base + (jbase + j) * K
            pltpu.make_async_copy(dst_hbm.at[pl.ds(off, K)], dis[ipar],
                                  dsems[ipar]).wait()
            pltpu.make_async_copy(ew_hbm.at[pl.ds(off, K)], ews[ipar],
                                  dsems[ipar]).wait()

        def scale(par, ipar):
            buf = bufs[par]
            ew_v = ews[ipar]

            def grp(g16, cc):
                wv = ew_v[pl.ds(g16 * 16, 16)]
                for r in range(16):
                    w = wv[r]
                    i = g16 * 16 + r
                    for t in range(D // 16):
                        buf[i, pl.ds(t * 16, 16)] = (
                            buf[i, pl.ds(t * 16, 16)] * w)
                return cc

            lax.fori_loop(0, K // 16, grp, 0)

        def fire_s(par, ipar):
            pltpu.async_copy(bufs[par], acc.at[dis[ipar]], ssems[par],
                             add=True)

        def wait_s(par, ipar):
            pltpu.make_async_copy(bufs[par], acc.at[dis[ipar]],
                                  ssems[par]).wait()

        fire_i(0, 0)
        fire_g(0, 0)
        fire_g(1, 1)
        fire_g(2, 2)

        def step(j, par, ipar):
            @pl.when(j < nchunks)
            def _():
                wait_g(j, par)
                wait_i(j, ipar)
                scale(par, ipar)
                fire_s(par, ipar)

                @pl.when(j >= 1)
                def _():
                    wait_s((par + 3) % 4, (ipar + 1) % 2)

                @pl.when(j + 3 < nchunks)
                def _():
                    fire_g(j + 3, (par + 3) % 4)

                @pl.when(j + 1 < nchunks)
                def _():
                    fire_i(j + 1, (ipar + 1) % 2)

        def quad(jj, carry):
            j = jj * 4
            step(j, 0, 0)
            step(j + 1, 1, 1)
            step(j + 2, 2, 0)
            step(j + 3, 3, 1)
            return carry

        lax.fori_loop(0, (nchunks + 3) // 4, quad, 0)
        wait_s((nchunks - 1) % 4, (nchunks - 1) % 2)

    emit_phase(PH_A, 0)
    pltpu.sync_copy(src_hbm.at[pl.ds(base + SI_LEN, PH_B * K)],
                    si_all.at[pl.ds(0, PH_B * K)])
    emit_phase(PH_B, PH_A)

    plsc.subcore_barrier()
    for r in range(DEG_CH // 24):
        pltpu.sync_copy(acc.at[pl.ds(s * DEG_CH + r * 24, 24)], zbuf)
        pltpu.sync_copy(zbuf, out_hbm.at[c, pl.ds(s * DEG_CH + r * 24, 24)])

    @pl.when(s == NS - 1)
    def _():
        pltpu.sync_copy(acc.at[pl.ds(N - 16, 16)], zbuf.at[pl.ds(0, 16)])
        pltpu.sync_copy(
            zbuf.at[pl.ds(0, 16)], out_hbm.at[c, pl.ds(N - 16, 16)]
        )


_BLK = 400
_GRID = N // _BLK


def _dot(a, b):
    return lax.dot_general(
        a, b, (((1,), (0,)), ((), ())),
        precision=lax.Precision.HIGHEST,
        preferred_element_type=jnp.float32,
    )


def _tc_first_body(d0, d1, x, w, dis_o, y_o):
    deg = d0[...] + d1[...] + 1.0
    dis = lax.rsqrt(deg)
    dis_o[...] = dis
    y_o[...] = _dot(x[...], w[...]) * dis


def _tc_mid_body(p0, p1, y, dis, b, w, y_o):
    z = jnp.maximum((p0[...] + p1[...] + y[...]) * dis[...] + b[...], 0.0)
    y_o[...] = _dot(z, w[...]) * dis[...]


def _tc_last_body(p0, p1, y, dis, b, o):
    o[...] = jnp.maximum((p0[...] + p1[...] + y[...]) * dis[...] + b[...], 0.0)


_row_spec = pl.BlockSpec((_BLK, D), lambda i: (i, 0))
_col_spec = pl.BlockSpec((_BLK, 1), lambda i: (i, 0))
_w_spec = pl.BlockSpec((D, D), lambda i: (0, 0))
_b_spec = pl.BlockSpec((1, D), lambda i: (0, 0))

_tc_first = pl.pallas_call(
    _tc_first_body,
    grid=(_GRID,),
    in_specs=[_col_spec, _col_spec, _row_spec, _w_spec],
    out_specs=[_col_spec, _row_spec],
    out_shape=[
        jax.ShapeDtypeStruct((N, 1), jnp.float32),
        jax.ShapeDtypeStruct((N, D), jnp.float32),
    ],
)

_tc_mid = pl.pallas_call(
    _tc_mid_body,
    grid=(_GRID,),
    in_specs=[_row_spec, _row_spec, _row_spec, _col_spec, _b_spec, _w_spec],
    out_specs=_row_spec,
    out_shape=jax.ShapeDtypeStruct((N, D), jnp.float32),
)

_tc_last = pl.pallas_call(
    _tc_last_body,
    grid=(_GRID,),
    in_specs=[_row_spec, _row_spec, _row_spec, _col_spec, _b_spec],
    out_specs=_row_spec,
    out_shape=jax.ShapeDtypeStruct((N, D), jnp.float32),
)


def kernel(x, edge_index, edge_weight, W0, b0, W1, b1, W2, b2):
    src = edge_index[0].astype(jnp.int32)
    dst = edge_index[1].astype(jnp.int32)
    ew = edge_weight.astype(jnp.float32)

    degp = _sc_deg(dst, ew)
    d0 = degp[:N].reshape(N, 1)
    d1 = degp[N:].reshape(N, 1)
    dis, y = _tc_first(d0, d1, x, W0)

    for b_, w_next in ((b0, W1), (b1, W2)):
        p = _sc_agg(y, src, dst, ew)
        y = _tc_mid(p[0], p[1], y, dis, b_.reshape(1, D), w_next)

    p = _sc_agg(y, src, dst, ew)
    return _tc_last(p[0], p[1], y, dis, b2.reshape(1, D))


# speedup vs baseline: 1.1350x; 1.1350x over previous
import functools

import jax
import jax.numpy as jnp
from jax import lax
from jax.experimental import pallas as pl
from jax.experimental.pallas import tpu as pltpu
from jax.experimental.pallas import tpu_sc as plsc

N = 10000
E = 320000
D = 128

NC = 2
NS = 16
NW = NC * NS
EPW = E // NW
K = 80
NCHUNK = EPW // K
DEG_CH = 624

_mesh = plsc.VectorSubcoreMesh(
    core_axis_name="c", subcore_axis_name="s", num_cores=NC, num_subcores=NS
)


@functools.partial(
    pl.kernel,
    out_type=jax.ShapeDtypeStruct((NC * N,), jnp.float32),
    mesh=_mesh,
    scratch_types=[
        pltpu.VMEM((EPW,), jnp.int32),
        pltpu.VMEM((EPW,), jnp.float32),
        pltpu.VMEM((K,), jnp.int32),
        pltpu.VMEM((K,), jnp.int32),
        pltpu.VMEM((DEG_CH + 16,), jnp.float32),
        pltpu.VMEM_SHARED((N,), jnp.float32),
        pltpu.SemaphoreType.DMA,
        pltpu.SemaphoreType.DMA,
    ],
)
def _sc_deg(dst_hbm, ew_hbm, out_hbm, di_all, ew_all, di_a, di_b, zbuf, acc,
            sem_a, sem_b):
    c = lax.axis_index("c")
    s = lax.axis_index("s")
    base = (c * NS + s) * EPW

    pltpu.sync_copy(dst_hbm.at[pl.ds(base, EPW)], di_all)
    pltpu.sync_copy(ew_hbm.at[pl.ds(base, EPW)], ew_all)

    for i in range((DEG_CH + 16) // 16):
        zbuf[pl.ds(i * 16, 16)] = jnp.zeros((16,), jnp.float32)
    pltpu.sync_copy(zbuf.at[pl.ds(0, DEG_CH)], acc.at[pl.ds(s * DEG_CH, DEG_CH)])

    @pl.when(s == NS - 1)
    def _():
        pltpu.sync_copy(zbuf.at[pl.ds(0, 16)], acc.at[pl.ds(N - 16, 16)])

    plsc.subcore_barrier()

    dis = (di_a, di_b)
    sems = (sem_a, sem_b)

    def stage_and_fire(j, par):
        di_v = dis[par]
        for g in range(K // 16):
            di_v[pl.ds(g * 16, 16)] = di_all[pl.ds(j * K + g * 16, 16)]
        pltpu.async_copy(ew_all.at[pl.ds(j * K, K)], acc.at[di_v],
                         sems[par], add=True)

    def drain(j, par):
        di_v = dis[par]
        pltpu.make_async_copy(ew_all.at[pl.ds(j * K, K)], acc.at[di_v],
                              sems[par]).wait()

    stage_and_fire(0, 0)

    def body2(jj, carry):
        stage_and_fire(jj * 2 + 1, 1)
        drain(jj * 2, 0)
        stage_and_fire(jj * 2 + 2, 0)
        drain(jj * 2 + 1, 1)
        return carry

    lax.fori_loop(0, (NCHUNK - 1) // 2, body2, 0)
    drain(NCHUNK - 1, 0)

    plsc.subcore_barrier()

    pltpu.sync_copy(acc.at[pl.ds(s * DEG_CH, DEG_CH)], zbuf.at[pl.ds(0, DEG_CH)])
    pltpu.sync_copy(
        zbuf.at[pl.ds(0, DEG_CH)],
        out_hbm.at[pl.ds(c * N + s * DEG_CH, DEG_CH)],
    )

    @pl.when(s == NS - 1)
    def _():
        pltpu.sync_copy(acc.at[pl.ds(N - 16, 16)], zbuf.at[pl.ds(0, 16)])
        pltpu.sync_copy(
            zbuf.at[pl.ds(0, 16)], out_hbm.at[pl.ds(c * N + N - 16, 16)]
        )


@functools.partial(
    pl.kernel,
    out_type=jax.ShapeDtypeStruct((NC, N, D), jnp.float32),
    mesh=_mesh,
    scratch_types=[
        pltpu.VMEM((EPW,), jnp.int32),
        pltpu.VMEM((K,), jnp.int32),
        pltpu.VMEM((K,), jnp.int32),
        pltpu.VMEM((K,), jnp.int32),
        pltpu.VMEM((K,), jnp.float32),
        pltpu.VMEM((K,), jnp.float32),
        pltpu.VMEM((K,), jnp.float32),
        pltpu.VMEM((K, D), jnp.float32),
        pltpu.VMEM((K, D), jnp.float32),
        pltpu.VMEM((K, D), jnp.float32),
        pltpu.VMEM((24, D), jnp.float32),
        pltpu.VMEM_SHARED((N, D), jnp.float32),
        pltpu.SemaphoreType.DMA,
        pltpu.SemaphoreType.DMA,
        pltpu.SemaphoreType.DMA,
        pltpu.SemaphoreType.DMA,
        pltpu.SemaphoreType.DMA,
        pltpu.SemaphoreType.DMA,
        pltpu.SemaphoreType.DMA,
        pltpu.SemaphoreType.DMA,
        pltpu.SemaphoreType.DMA,
    ],
)
def _sc_agg(y_hbm, src_hbm, dst_hbm, ew_hbm, out_hbm, si_all,
            di_a, di_b, di_c, ew_a, ew_b, ew_c, buf_a, buf_b, buf_c,
            zbuf, acc, gs_a, gs_b, gs_c, ss_a, ss_b, ss_c,
            ds_a, ds_b, ds_c):
    c = lax.axis_index("c")
    s = lax.axis_index("s")
    base = (c * NS + s) * EPW

    pltpu.sync_copy(src_hbm.at[pl.ds(base, EPW)], si_all)

    def zrow(i, carry):
        for t in range(D // 16):
            zbuf[i, pl.ds(t * 16, 16)] = jnp.zeros((16,), jnp.float32)
        return carry

    lax.fori_loop(0, 24, zrow, 0)
    for r in range(DEG_CH // 24):
        pltpu.sync_copy(zbuf, acc.at[pl.ds(s * DEG_CH + r * 24, 24)])

    @pl.when(s == NS - 1)
    def _():
        pltpu.sync_copy(zbuf.at[pl.ds(0, 16)], acc.at[pl.ds(N - 16, 16)])

    plsc.subcore_barrier()

    bufs = (buf_a, buf_b, buf_c)
    dis = (di_a, di_b, di_c)
    ews = (ew_a, ew_b, ew_c)
    gsems = (gs_a, gs_b, gs_c)
    ssems = (ss_a, ss_b, ss_c)
    dsems = (ds_a, ds_b, ds_c)
    H = K // 2

    def fire_gather(j, par):
        pltpu.async_copy(
            y_hbm.at[si_all.at[pl.ds(j * K, H)]],
            bufs[par].at[pl.ds(0, H)], gsems[par]
        )
        pltpu.async_copy(
            y_hbm.at[si_all.at[pl.ds(j * K + H, H)]],
            bufs[par].at[pl.ds(H, H)], gsems[par]
        )
        pltpu.async_copy(dst_hbm.at[pl.ds(base + j * K, K)], dis[par],
                         dsems[par])
        pltpu.async_copy(ew_hbm.at[pl.ds(base + j * K, K)], ews[par],
                         dsems[par])

    def wait_gather(j, par):
        pltpu.make_async_copy(
            y_hbm.at[si_all.at[pl.ds(j * K, K)]], bufs[par], gsems[par]
        ).wait()
        pltpu.make_async_copy(dst_hbm.at[pl.ds(base + j * K, K)], dis[par],
                              dsems[par]).wait()
        pltpu.make_async_copy(ew_hbm.at[pl.ds(base + j * K, K)], ews[par],
                              dsems[par]).wait()

    def scale(j, par):
        buf = bufs[par]

        ew_v = ews[par]

        def grp(g16, cc):
            wv = ew_v[pl.ds(g16 * 16, 16)]
            for r in range(16):
                w = wv[r]
                i = g16 * 16 + r
                for t in range(D // 16):
                    buf[i, pl.ds(t * 16, 16)] = buf[i, pl.ds(t * 16, 16)] * w
            return cc

        lax.fori_loop(0, K // 16, grp, 0)

    def fire_scatter(j, par):
        pltpu.async_copy(bufs[par], acc.at[dis[par]], ssems[par], add=True)

    def wait_scatter(par):
        pltpu.make_async_copy(bufs[par], acc.at[dis[par]], ssems[par]).wait()

    fire_gather(0, 0)
    fire_gather(1, 1)

    def step(j, par, first, fire_ahead):
        wait_gather(j, par)
        scale(j, par)
        fire_scatter(j, par)
        if not first:
            wait_scatter((par + 2) % 3)
        if fire_ahead:
            fire_gather(j + 2, (par + 2) % 3)

    step(0, 0, True, True)
    step(1, 1, False, True)
    step(2, 2, False, True)

    def body3(jj, carry):
        j = jj * 3
        step(j, 0, False, True)
        step(j + 1, 1, False, True)
        step(j + 2, 2, False, True)
        return carry

    lax.fori_loop(1, (NCHUNK - 2) // 3, body3, 0)
    step(NCHUNK - 2, 0, False, False)
    step(NCHUNK - 1, 1, False, False)
    wait_scatter(1)

    plsc.subcore_barrier()
    for r in range(DEG_CH // 24):
        pltpu.sync_copy(acc.at[pl.ds(s * DEG_CH + r * 24, 24)], zbuf)
        pltpu.sync_copy(zbuf, out_hbm.at[c, pl.ds(s * DEG_CH + r * 24, 24)])

    @pl.when(s == NS - 1)
    def _():
        pltpu.sync_copy(acc.at[pl.ds(N - 16, 16)], zbuf.at[pl.ds(0, 16)])
        pltpu.sync_copy(
            zbuf.at[pl.ds(0, 16)], out_hbm.at[c, pl.ds(N - 16, 16)]
        )


_BLK = 1000
_GRID = N // _BLK


def _dot(a, b):
    return lax.dot_general(
        a, b, (((1,), (0,)), ((), ())),
        precision=lax.Precision.HIGHEST,
        preferred_element_type=jnp.float32,
    )


def _tc_first_body(d0, d1, x, w, dis_o, y_o):
    deg = d0[...] + d1[...] + 1.0
    dis = lax.rsqrt(deg)
    dis_o[...] = dis
    y_o[...] = _dot(x[...], w[...]) * dis


def _tc_mid_body(p0, p1, y, dis, b, w, y_o):
    z = jnp.maximum((p0[...] + p1[...] + y[...]) * dis[...] + b[...], 0.0)
    y_o[...] = _dot(z, w[...]) * dis[...]


def _tc_last_body(p0, p1, y, dis, b, o):
    o[...] = jnp.maximum((p0[...] + p1[...] + y[...]) * dis[...] + b[...], 0.0)


_row_spec = pl.BlockSpec((_BLK, D), lambda i: (i, 0))
_col_spec = pl.BlockSpec((_BLK, 1), lambda i: (i, 0))
_w_spec = pl.BlockSpec((D, D), lambda i: (0, 0))
_b_spec = pl.BlockSpec((1, D), lambda i: (0, 0))

_tc_first = pl.pallas_call(
    _tc_first_body,
    grid=(_GRID,),
    in_specs=[_col_spec, _col_spec, _row_spec, _w_spec],
    out_specs=[_col_spec, _row_spec],
    out_shape=[
        jax.ShapeDtypeStruct((N, 1), jnp.float32),
        jax.ShapeDtypeStruct((N, D), jnp.float32),
    ],
)

_tc_mid = pl.pallas_call(
    _tc_mid_body,
    grid=(_GRID,),
    in_specs=[_row_spec, _row_spec, _row_spec, _col_spec, _b_spec, _w_spec],
    out_specs=_row_spec,
    out_shape=jax.ShapeDtypeStruct((N, D), jnp.float32),
)

_tc_last = pl.pallas_call(
    _tc_last_body,
    grid=(_GRID,),
    in_specs=[_row_spec, _row_spec, _row_spec, _col_spec, _b_spec],
    out_specs=_row_spec,
    out_shape=jax.ShapeDtypeStruct((N, D), jnp.float32),
)


def kernel(x, edge_index, edge_weight, W0, b0, W1, b1, W2, b2):
    src = edge_index[0].astype(jnp.int32)
    dst = edge_index[1].astype(jnp.int32)
    ew = edge_weight.astype(jnp.float32)

    degp = _sc_deg(dst, ew)
    d0 = degp[:N].reshape(N, 1)
    d1 = degp[N:].reshape(N, 1)
    dis, y = _tc_first(d0, d1, x, W0)

    for b_, w_next in ((b0, W1), (b1, W2)):
        p = _sc_agg(y, src, dst, ew)
        y = _tc_mid(p[0], p[1], y, dis, b_.reshape(1, D), w_next)

    p = _sc_agg(y, src, dst, ew)
    return _tc_last(p[0], p[1], y, dis, b2.reshape(1, D))


# speedup vs baseline: 1.1617x; 1.0235x over previous
import functools

import jax
import jax.numpy as jnp
from jax import lax
from jax.experimental import pallas as pl
from jax.experimental.pallas import tpu as pltpu
from jax.experimental.pallas import tpu_sc as plsc

N = 10000
E = 320000
D = 128

NC = 2
NS = 16
NW = NC * NS
EPW = E // NW
K = 80
NCHUNK = EPW // K
DEG_CH = 624

_mesh = plsc.VectorSubcoreMesh(
    core_axis_name="c", subcore_axis_name="s", num_cores=NC, num_subcores=NS
)


@functools.partial(
    pl.kernel,
    out_type=jax.ShapeDtypeStruct((NC * N,), jnp.float32),
    mesh=_mesh,
    scratch_types=[
        pltpu.VMEM((EPW,), jnp.int32),
        pltpu.VMEM((EPW,), jnp.float32),
        pltpu.VMEM((K,), jnp.int32),
        pltpu.VMEM((K,), jnp.int32),
        pltpu.VMEM((DEG_CH + 16,), jnp.float32),
        pltpu.VMEM_SHARED((N,), jnp.float32),
        pltpu.SemaphoreType.DMA,
        pltpu.SemaphoreType.DMA,
    ],
)
def _sc_deg(dst_hbm, ew_hbm, out_hbm, di_all, ew_all, di_a, di_b, zbuf, acc,
            sem_a, sem_b):
    c = lax.axis_index("c")
    s = lax.axis_index("s")
    base = (c * NS + s) * EPW

    pltpu.sync_copy(dst_hbm.at[pl.ds(base, EPW)], di_all)
    pltpu.sync_copy(ew_hbm.at[pl.ds(base, EPW)], ew_all)

    for i in range((DEG_CH + 16) // 16):
        zbuf[pl.ds(i * 16, 16)] = jnp.zeros((16,), jnp.float32)
    pltpu.sync_copy(zbuf.at[pl.ds(0, DEG_CH)], acc.at[pl.ds(s * DEG_CH, DEG_CH)])

    @pl.when(s == NS - 1)
    def _():
        pltpu.sync_copy(zbuf.at[pl.ds(0, 16)], acc.at[pl.ds(N - 16, 16)])

    plsc.subcore_barrier()

    dis = (di_a, di_b)
    sems = (sem_a, sem_b)

    def stage_and_fire(j, par):
        di_v = dis[par]
        for g in range(K // 16):
            di_v[pl.ds(g * 16, 16)] = di_all[pl.ds(j * K + g * 16, 16)]
        pltpu.async_copy(ew_all.at[pl.ds(j * K, K)], acc.at[di_v],
                         sems[par], add=True)

    def drain(j, par):
        di_v = dis[par]
        pltpu.make_async_copy(ew_all.at[pl.ds(j * K, K)], acc.at[di_v],
                              sems[par]).wait()

    stage_and_fire(0, 0)

    def body2(jj, carry):
        stage_and_fire(jj * 2 + 1, 1)
        drain(jj * 2, 0)
        stage_and_fire(jj * 2 + 2, 0)
        drain(jj * 2 + 1, 1)
        return carry

    lax.fori_loop(0, (NCHUNK - 1) // 2, body2, 0)
    drain(NCHUNK - 1, 0)

    plsc.subcore_barrier()

    pltpu.sync_copy(acc.at[pl.ds(s * DEG_CH, DEG_CH)], zbuf.at[pl.ds(0, DEG_CH)])
    pltpu.sync_copy(
        zbuf.at[pl.ds(0, DEG_CH)],
        out_hbm.at[pl.ds(c * N + s * DEG_CH, DEG_CH)],
    )

    @pl.when(s == NS - 1)
    def _():
        pltpu.sync_copy(acc.at[pl.ds(N - 16, 16)], zbuf.at[pl.ds(0, 16)])
        pltpu.sync_copy(
            zbuf.at[pl.ds(0, 16)], out_hbm.at[pl.ds(c * N + N - 16, 16)]
        )


@functools.partial(
    pl.kernel,
    out_type=jax.ShapeDtypeStruct((NC, N, D), jnp.float32),
    mesh=_mesh,
    scratch_types=[
        pltpu.VMEM((EPW,), jnp.int32),
        pltpu.VMEM((K,), jnp.int32),
        pltpu.VMEM((K,), jnp.int32),
        pltpu.VMEM((K,), jnp.int32),
        pltpu.VMEM((K,), jnp.float32),
        pltpu.VMEM((K,), jnp.float32),
        pltpu.VMEM((K,), jnp.float32),
        pltpu.VMEM((K, D), jnp.float32),
        pltpu.VMEM((K, D), jnp.float32),
        pltpu.VMEM((K, D), jnp.float32),
        pltpu.VMEM((24, D), jnp.float32),
        pltpu.VMEM_SHARED((N, D), jnp.float32),
        pltpu.SemaphoreType.DMA,
        pltpu.SemaphoreType.DMA,
        pltpu.SemaphoreType.DMA,
        pltpu.SemaphoreType.DMA,
        pltpu.SemaphoreType.DMA,
        pltpu.SemaphoreType.DMA,
        pltpu.SemaphoreType.DMA,
        pltpu.SemaphoreType.DMA,
        pltpu.SemaphoreType.DMA,
    ],
)
def _sc_agg(y_hbm, src_hbm, dst_hbm, ew_hbm, out_hbm, si_all,
            di_a, di_b, di_c, ew_a, ew_b, ew_c, buf_a, buf_b, buf_c,
            zbuf, acc, gs_a, gs_b, gs_c, ss_a, ss_b, ss_c,
            ds_a, ds_b, ds_c):
    c = lax.axis_index("c")
    s = lax.axis_index("s")
    base = (c * NS + s) * EPW

    pltpu.sync_copy(src_hbm.at[pl.ds(base, EPW)], si_all)

    def zrow(i, carry):
        for t in range(D // 16):
            zbuf[i, pl.ds(t * 16, 16)] = jnp.zeros((16,), jnp.float32)
        return carry

    lax.fori_loop(0, 24, zrow, 0)
    for r in range(DEG_CH // 24):
        pltpu.sync_copy(zbuf, acc.at[pl.ds(s * DEG_CH + r * 24, 24)])

    @pl.when(s == NS - 1)
    def _():
        pltpu.sync_copy(zbuf.at[pl.ds(0, 16)], acc.at[pl.ds(N - 16, 16)])

    plsc.subcore_barrier()

    bufs = (buf_a, buf_b, buf_c)
    dis = (di_a, di_b, di_c)
    ews = (ew_a, ew_b, ew_c)
    gsems = (gs_a, gs_b, gs_c)
    ssems = (ss_a, ss_b, ss_c)
    dsems = (ds_a, ds_b, ds_c)
    H = K // 2

    def fire_gather(j, par):
        pltpu.async_copy(
            y_hbm.at[si_all.at[pl.ds(j * K, H)]],
            bufs[par].at[pl.ds(0, H)], gsems[par]
        )
        pltpu.async_copy(
            y_hbm.at[si_all.at[pl.ds(j * K + H, H)]],
            bufs[par].at[pl.ds(H, H)], gsems[par]
        )
        pltpu.async_copy(dst_hbm.at[pl.ds(base + j * K, K)], dis[par],
                         dsems[par])
        pltpu.async_copy(ew_hbm.at[pl.ds(base + j * K, K)], ews[par],
                         dsems[par])

    def wait_gather(j, par):
        pltpu.make_async_copy(
            y_hbm.at[si_all.at[pl.ds(j * K, K)]], bufs[par], gsems[par]
        ).wait()
        pltpu.make_async_copy(dst_hbm.at[pl.ds(base + j * K, K)], dis[par],
                              dsems[par]).wait()
        pltpu.make_async_copy(ew_hbm.at[pl.ds(base + j * K, K)], ews[par],
                              dsems[par]).wait()

    def scale(j, par):
        buf = bufs[par]

        ew_v = ews[par]

        def grp(g16, cc):
            wv = ew_v[pl.ds(g16 * 16, 16)]
            for r in range(16):
                w = wv[r]
                i = g16 * 16 + r
                for t in range(D // 16):
                    buf[i, pl.ds(t * 16, 16)] = buf[i, pl.ds(t * 16, 16)] * w
            return cc

        lax.fori_loop(0, K // 16, grp, 0)

    def fire_scatter(j, par):
        pltpu.async_copy(bufs[par], acc.at[dis[par]], ssems[par], add=True)

    def wait_scatter(par):
        pltpu.make_async_copy(bufs[par], acc.at[dis[par]], ssems[par]).wait()

    fire_gather(0, 0)
    fire_gather(1, 1)

    def step(j, par, first, fire_ahead):
        wait_gather(j, par)
        scale(j, par)
        fire_scatter(j, par)
        if not first:
            wait_scatter((par + 2) % 3)
        if fire_ahead:
            fire_gather(j + 2, (par + 2) % 3)

    step(0, 0, True, True)
    step(1, 1, False, True)
    step(2, 2, False, True)

    def body3(jj, carry):
        j = jj * 3
        step(j, 0, False, True)
        step(j + 1, 1, False, True)
        step(j + 2, 2, False, True)
        return carry

    lax.fori_loop(1, (NCHUNK - 2) // 3, body3, 0)
    step(NCHUNK - 2, 0, False, False)
    step(NCHUNK - 1, 1, False, False)
    wait_scatter(1)

    plsc.subcore_barrier()
    for r in range(DEG_CH // 24):
        pltpu.sync_copy(acc.at[pl.ds(s * DEG_CH + r * 24, 24)], zbuf)
        pltpu.sync_copy(zbuf, out_hbm.at[c, pl.ds(s * DEG_CH + r * 24, 24)])

    @pl.when(s == NS - 1)
    def _():
        pltpu.sync_copy(acc.at[pl.ds(N - 16, 16)], zbuf.at[pl.ds(0, 16)])
        pltpu.sync_copy(
            zbuf.at[pl.ds(0, 16)], out_hbm.at[c, pl.ds(N - 16, 16)]
        )


_BLK = 2000
_GRID = N // _BLK


def _dot(a, b):
    return lax.dot_general(
        a, b, (((1,), (0,)), ((), ())),
        precision=lax.Precision.HIGHEST,
        preferred_element_type=jnp.float32,
    )


def _tc_first_body(d0, d1, x, w, dis_o, y_o):
    deg = d0[...] + d1[...] + 1.0
    dis = lax.rsqrt(deg)
    dis_o[...] = dis
    y_o[...] = _dot(x[...], w[...]) * dis


def _tc_mid_body(p0, p1, y, dis, b, w, y_o):
    z = jnp.maximum((p0[...] + p1[...] + y[...]) * dis[...] + b[...], 0.0)
    y_o[...] = _dot(z, w[...]) * dis[...]


def _tc_last_body(p0, p1, y, dis, b, o):
    o[...] = jnp.maximum((p0[...] + p1[...] + y[...]) * dis[...] + b[...], 0.0)


_row_spec = pl.BlockSpec((_BLK, D), lambda i: (i, 0))
_col_spec = pl.BlockSpec((_BLK, 1), lambda i: (i, 0))
_w_spec = pl.BlockSpec((D, D), lambda i: (0, 0))
_b_spec = pl.BlockSpec((1, D), lambda i: (0, 0))

_tc_first = pl.pallas_call(
    _tc_first_body,
    grid=(_GRID,),
    in_specs=[_col_spec, _col_spec, _row_spec, _w_spec],
    out_specs=[_col_spec, _row_spec],
    out_shape=[
        jax.ShapeDtypeStruct((N, 1), jnp.float32),
        jax.ShapeDtypeStruct((N, D), jnp.float32),
    ],
)

_tc_mid = pl.pallas_call(
    _tc_mid_body,
    grid=(_GRID,),
    in_specs=[_row_spec, _row_spec, _row_spec, _col_spec, _b_spec, _w_spec],
    out_specs=_row_spec,
    out_shape=jax.ShapeDtypeStruct((N, D), jnp.float32),
)

_tc_last = pl.pallas_call(
    _tc_last_body,
    grid=(_GRID,),
    in_specs=[_row_spec, _row_spec, _row_spec, _col_spec, _b_spec],
    out_specs=_row_spec,
    out_shape=jax.ShapeDtypeStruct((N, D), jnp.float32),
)


def kernel(x, edge_index, edge_weight, W0, b0, W1, b1, W2, b2):
    src = edge_index[0].astype(jnp.int32)
    dst = edge_index[1].astype(jnp.int32)
    ew = edge_weight.astype(jnp.float32)

    degp = _sc_deg(dst, ew)
    d0 = degp[:N].reshape(N, 1)
    d1 = degp[N:].reshape(N, 1)
    dis, y = _tc_first(d0, d1, x, W0)

    for b_, w_next in ((b0, W1), (b1, W2)):
        p = _sc_agg(y, src, dst, ew)
        y = _tc_mid(p[0], p[1], y, dis, b_.reshape(1, D), w_next)

    p = _sc_agg(y, src, dst, ew)
    return _tc_last(p[0], p[1], y, dis, b2.reshape(1, D))
